# baseline (speedup 1.0000x reference)
import functools
import math

import jax
import jax.numpy as jnp
from jax import lax
from jax.experimental import pallas as pl
from jax.experimental.pallas import tpu as pltpu
from jax.experimental.pallas import tpu_sc as plsc

_NC = 2
_NS = 16
_LANES = 16
_NW = _NC * _NS

_RB = 8192


def _sc_gather_body(x_hbm, e0_hbm, e1_hbm, mw_hbm, bw_hbm, out_hbm,
                    x_v, e0_v, e1_v, mw_v, bw_v, xx_v, sem):
    chunk = x_v.shape[0]
    wid = lax.axis_index("s") * _NC + lax.axis_index("c")
    base = wid * chunk
    copies = [
        pltpu.make_async_copy(mw_hbm, mw_v, sem),
        pltpu.make_async_copy(bw_hbm, bw_v, sem),
        pltpu.make_async_copy(x_hbm.at[pl.ds(base, chunk)], x_v, sem),
        pltpu.make_async_copy(e0_hbm.at[pl.ds(base, chunk)], e0_v, sem),
        pltpu.make_async_copy(e1_hbm.at[pl.ds(base, chunk)], e1_v, sem),
    ]
    for c in copies:
        c.start()
    for c in copies:
        c.wait()

    @plsc.parallel_loop(0, chunk // _LANES, unroll=16)
    def _step(i):
        sl = pl.ds(i * _LANES, _LANES)
        i0 = e0_v[sl]
        i1 = e1_v[sl]
        m = plsc.load_gather(mw_v, [i0]) + plsc.load_gather(mw_v, [i1])
        b = plsc.load_gather(bw_v, [i0]) + plsc.load_gather(bw_v, [i1])
        xx_v[sl] = m * x_v[sl] + b

    pltpu.sync_copy(xx_v, out_hbm.at[pl.ds(base, chunk)])


def _sc_gather(x_flat, e0, e1, mw, bw):
    t = x_flat.shape[0]
    chunk = t // _NW
    e = mw.shape[0]
    kern = functools.partial(
        pl.kernel,
        mesh=plsc.VectorSubcoreMesh(core_axis_name="c", subcore_axis_name="s"),
        out_type=jax.ShapeDtypeStruct((t,), jnp.float32),
        compiler_params=pltpu.CompilerParams(needs_layout_passes=False),
        scratch_types=[
            pltpu.VMEM((chunk,), jnp.float32),
            pltpu.VMEM((chunk,), jnp.int32),
            pltpu.VMEM((chunk,), jnp.int32),
            pltpu.VMEM((e,), jnp.float32),
            pltpu.VMEM((e,), jnp.float32),
            pltpu.VMEM((chunk,), jnp.float32),
            pltpu.SemaphoreType.DMA,
        ],
    )(_sc_gather_body)
    return kern(x_flat, e0, e1, mw, bw)


def _rbf_body(xx_ref, means_ref, stds_ref, out_ref):
    std = jnp.abs(stds_ref[...]) + 0.01
    inv = 1.0 / std
    ls = jnp.log(inv * (1.0 / math.sqrt(2.0 * math.pi)))
    xxt = xx_ref[...].T
    for g in range(xxt.shape[1]):
        col = xxt[:, g:g + 1]
        z = (col - means_ref[...]) * inv
        out_ref[g * 128:(g + 1) * 128, :] = jnp.exp(ls - 0.5 * (z * z))


def _rbf(xx, means, stds):
    t = xx.shape[0]
    k = means.shape[-1]
    return pl.pallas_call(
        _rbf_body,
        grid=(t // _RB,),
        in_specs=[
            pl.BlockSpec((_RB // 128, 128), lambda g: (g, 0)),
            pl.BlockSpec((1, k), lambda g: (0, 0)),
            pl.BlockSpec((1, k), lambda g: (0, 0)),
        ],
        out_specs=pl.BlockSpec((_RB, k), lambda g: (g, 0)),
        out_shape=jax.ShapeDtypeStruct((t, k), jnp.float32),
    )(xx.reshape(t // 128, 128), means, stds)


def kernel(x, edge_types, means, stds, mul_w, bias_w):
    out_shape = x.shape
    k = means.shape[-1]
    t = x.size
    x_flat = x.reshape(t).astype(jnp.float32)
    et = edge_types.reshape(t, 2).astype(jnp.int32)
    e0 = et[:, 0]
    e1 = et[:, 1]
    mw = mul_w.reshape(-1).astype(jnp.float32)
    bw = bias_w.reshape(-1).astype(jnp.float32)
    xx = _sc_gather(x_flat, e0, e1, mw, bw)
    out = _rbf(xx, means.astype(jnp.float32), stds.astype(jnp.float32))
    return out.reshape(out_shape + (k,)).astype(means.dtype)


# speedup vs baseline: 1.0805x; 1.0805x over previous
import functools
import math

import jax
import jax.numpy as jnp
from jax import lax
from jax.experimental import pallas as pl
from jax.experimental.pallas import tpu as pltpu
from jax.experimental.pallas import tpu_sc as plsc

_NC = 2
_NS = 16
_LANES = 16
_NW = _NC * _NS

_RB = 32768


def _sc_gather_body(x_hbm, e0_hbm, e1_hbm, mw_hbm, bw_hbm, out_hbm,
                    x_v, e0_v, e1_v, mw_v, bw_v, xx_v, sem):
    chunk = x_v.shape[0]
    wid = lax.axis_index("s") * _NC + lax.axis_index("c")
    base = wid * chunk
    copies = [
        pltpu.make_async_copy(mw_hbm, mw_v, sem),
        pltpu.make_async_copy(bw_hbm, bw_v, sem),
        pltpu.make_async_copy(x_hbm.at[pl.ds(base, chunk)], x_v, sem),
        pltpu.make_async_copy(e0_hbm.at[pl.ds(base, chunk)], e0_v, sem),
        pltpu.make_async_copy(e1_hbm.at[pl.ds(base, chunk)], e1_v, sem),
    ]
    for c in copies:
        c.start()
    for c in copies:
        c.wait()

    @plsc.parallel_loop(0, chunk // _LANES, unroll=16)
    def _step(i):
        sl = pl.ds(i * _LANES, _LANES)
        i0 = e0_v[sl]
        i1 = e1_v[sl]
        m = plsc.load_gather(mw_v, [i0]) + plsc.load_gather(mw_v, [i1])
        b = plsc.load_gather(bw_v, [i0]) + plsc.load_gather(bw_v, [i1])
        xx_v[sl] = m * x_v[sl] + b

    pltpu.sync_copy(xx_v, out_hbm.at[pl.ds(base, chunk)])


def _sc_gather(x_flat, e0, e1, mw, bw):
    t = x_flat.shape[0]
    chunk = t // _NW
    e = mw.shape[0]
    kern = functools.partial(
        pl.kernel,
        mesh=plsc.VectorSubcoreMesh(core_axis_name="c", subcore_axis_name="s"),
        out_type=jax.ShapeDtypeStruct((t,), jnp.float32),
        compiler_params=pltpu.CompilerParams(needs_layout_passes=False),
        scratch_types=[
            pltpu.VMEM((chunk,), jnp.float32),
            pltpu.VMEM((chunk,), jnp.int32),
            pltpu.VMEM((chunk,), jnp.int32),
            pltpu.VMEM((e,), jnp.float32),
            pltpu.VMEM((e,), jnp.float32),
            pltpu.VMEM((chunk,), jnp.float32),
            pltpu.SemaphoreType.DMA,
        ],
    )(_sc_gather_body)
    return kern(x_flat, e0, e1, mw, bw)


def _rbf_body(xx_ref, means_ref, stds_ref, out_ref):
    std = jnp.abs(stds_ref[...]) + 0.01
    inv = 1.0 / std
    ls = jnp.log(inv * (1.0 / math.sqrt(2.0 * math.pi)))
    xxt = xx_ref[...].T
    for g in range(xxt.shape[1]):
        col = xxt[:, g:g + 1]
        z = (col - means_ref[...]) * inv
        out_ref[g * 128:(g + 1) * 128, :] = jnp.exp(ls - 0.5 * (z * z))


def _rbf(xx, means, stds):
    t = xx.shape[0]
    k = means.shape[-1]
    return pl.pallas_call(
        _rbf_body,
        grid=(t // _RB,),
        in_specs=[
            pl.BlockSpec((_RB // 128, 128), lambda g: (g, 0)),
            pl.BlockSpec((1, k), lambda g: (0, 0)),
            pl.BlockSpec((1, k), lambda g: (0, 0)),
        ],
        out_specs=pl.BlockSpec((_RB, k), lambda g: (g, 0)),
        out_shape=jax.ShapeDtypeStruct((t, k), jnp.float32),
    )(xx.reshape(t // 128, 128), means, stds)


def kernel(x, edge_types, means, stds, mul_w, bias_w):
    out_shape = x.shape
    k = means.shape[-1]
    t = x.size
    x_flat = x.reshape(t).astype(jnp.float32)
    et = edge_types.reshape(t, 2).astype(jnp.int32)
    e0 = et[:, 0]
    e1 = et[:, 1]
    mw = mul_w.reshape(-1).astype(jnp.float32)
    bw = bias_w.reshape(-1).astype(jnp.float32)
    xx = _sc_gather(x_flat, e0, e1, mw, bw)
    out = _rbf(xx, means.astype(jnp.float32), stds.astype(jnp.float32))
    return out.reshape(out_shape + (k,)).astype(means.dtype)
